# Initial kernel scaffold; baseline (speedup 1.0000x reference)
import jax
import jax.numpy as jnp
from jax.experimental import pallas as pl


def kernel(pos, x, W1, g1, b1, kpts, Wkp, gk, bk, W2, g2, b2):
    raise NotImplementedError("write your pallas kernel here")


# speedup vs baseline: 2.3206x; 2.3206x over previous
import functools

import jax
import jax.numpy as jnp
from jax.experimental import pallas as pl
from jax.experimental.pallas import tpu as pltpu

_EPS = 1e-5
_NEG = 0.1
_INFL = 0.08
_KNN = 16
_KPTS = 15


def _leaky(v):
    return jnp.where(v >= 0, v, _NEG * v)


def _bn_act(v, g, b):
    m = jnp.mean(v, axis=0, keepdims=True)
    var = jnp.mean((v - m) ** 2, axis=0, keepdims=True)
    return _leaky((v - m) / jnp.sqrt(var + _EPS) * g + b)


def _unary1_body(x_ref, w_ref, g_ref, b_ref, o_ref):
    y = jnp.dot(x_ref[...], w_ref[...], preferred_element_type=jnp.float32)
    o_ref[...] = _bn_act(y, g_ref[...], b_ref[...])


def _knn_agg_body(q_rows, npad, big, xyzq_ref, xyzt_ref, rhi_ref, rlo_ref,
                  kptst_ref, wkp_ref, o_ref):
    qx = xyzq_ref[:, 0:1]
    qy = xyzq_ref[:, 1:2]
    qz = xyzq_ref[:, 2:3]
    sx = xyzt_ref[0:1, :]
    sy = xyzt_ref[1:2, :]
    sz = xyzt_ref[2:3, :]
    dx = qx - sx
    dy = qy - sy
    dz = qz - sz
    d2 = dx * dx + dy * dy + dz * dz
    iota = jax.lax.broadcasted_iota(jnp.int32, (q_rows, npad), 1)
    kx = kptst_ref[0:1, :]
    ky = kptst_ref[1:2, :]
    kz = kptst_ref[2:3, :]
    accs = tuple(jnp.zeros((q_rows, 32), jnp.float32) for _ in range(_KPTS))

    def step(_, carry):
        d2, accs = carry
        m = jnp.min(d2, axis=1, keepdims=True)
        amin = jnp.min(jnp.where(d2 == m, iota, npad), axis=1, keepdims=True)
        eq = iota == amin
        selx = jnp.sum(jnp.where(eq, sx, 0.0), axis=1, keepdims=True)
        sely = jnp.sum(jnp.where(eq, sy, 0.0), axis=1, keepdims=True)
        selz = jnp.sum(jnp.where(eq, sz, 0.0), axis=1, keepdims=True)
        feat = (jnp.dot(eq.astype(jnp.bfloat16), rhi_ref[...],
                        preferred_element_type=jnp.float32)
                + jnp.dot(eq.astype(jnp.float32), rlo_ref[...],
                          preferred_element_type=jnp.float32))
        d2 = jnp.where(eq, big, d2)
        rx = selx - qx
        ry = sely - qy
        rz = selz - qz
        sq = (rx - kx) ** 2 + (ry - ky) ** 2 + (rz - kz) ** 2
        w = jnp.maximum(1.0 - jnp.sqrt(sq + 1e-12) / _INFL, 0.0)
        accs = tuple(accs[k] + w[:, k:k + 1] * feat for k in range(_KPTS))
        return d2, accs

    _, accs = jax.lax.fori_loop(0, _KNN, step, (d2, accs))
    out = jnp.zeros((q_rows, 32), jnp.float32)
    for k in range(_KPTS):
        out = out + jnp.dot(accs[k], wkp_ref[k],
                            preferred_element_type=jnp.float32)
    o_ref[...] = out


def _post_body(a_ref, gk_ref, bk_ref, w2_ref, g2_ref, b2_ref, xs_ref, o_ref):
    h = _bn_act(a_ref[...], gk_ref[...], bk_ref[...])
    y = jnp.dot(h, w2_ref[...], preferred_element_type=jnp.float32)
    o_ref[...] = _bn_act(y, g2_ref[...], b2_ref[...]) + xs_ref[...]


@jax.jit
def kernel(pos, x, W1, g1, b1, kpts, Wkp, gk, bk, W2, g2, b2):
    n, d_in = x.shape
    d2ch = W1.shape[1]
    xyz = pos[:, 1:4]

    h1 = pl.pallas_call(
        _unary1_body,
        out_shape=jax.ShapeDtypeStruct((n, d2ch), jnp.float32),
    )(x, W1, g1.reshape(1, -1), b1.reshape(1, -1))

    q_rows = 128
    npad = ((n + q_rows - 1) // q_rows) * q_rows
    pad = npad - n
    padv = 1e6
    xyz_pad = jnp.concatenate(
        [xyz, jnp.full((pad, 3), padv, jnp.float32)], axis=0)
    h1_pad = jnp.concatenate([h1, jnp.zeros((pad, d2ch), jnp.float32)], axis=0)
    r_hi = h1_pad.astype(jnp.bfloat16)
    r_lo = h1_pad - r_hi.astype(jnp.float32)
    xyzt = xyz_pad.T
    kptst = kpts.T
    grid = npad // q_rows

    body = functools.partial(_knn_agg_body, q_rows, npad, 1e30)
    agg = pl.pallas_call(
        body,
        grid=(grid,),
        in_specs=[
            pl.BlockSpec((q_rows, 3), lambda i: (i, 0)),
            pl.BlockSpec((3, npad), lambda i: (0, 0)),
            pl.BlockSpec((npad, d2ch), lambda i: (0, 0)),
            pl.BlockSpec((npad, d2ch), lambda i: (0, 0)),
            pl.BlockSpec((3, _KPTS), lambda i: (0, 0)),
            pl.BlockSpec((_KPTS, d2ch, d2ch), lambda i: (0, 0, 0)),
        ],
        out_specs=pl.BlockSpec((q_rows, d2ch), lambda i: (i, 0)),
        out_shape=jax.ShapeDtypeStruct((npad, d2ch), jnp.float32),
        compiler_params=pltpu.CompilerParams(
            dimension_semantics=("arbitrary",)),
    )(xyz_pad, xyzt, r_hi, r_lo, kptst, Wkp)
    agg = agg[:n]

    out = pl.pallas_call(
        _post_body,
        out_shape=jax.ShapeDtypeStruct((n, d_in), jnp.float32),
    )(agg, gk.reshape(1, -1), bk.reshape(1, -1), W2, g2.reshape(1, -1),
      b2.reshape(1, -1), x)
    return out


# speedup vs baseline: 2.5490x; 1.0984x over previous
import functools

import jax
import jax.numpy as jnp
from jax.experimental import pallas as pl
from jax.experimental.pallas import tpu as pltpu

_EPS = 1e-5
_NEG = 0.1
_INFL = 0.08
_KNN = 16
_KPTS = 15


def _leaky(v):
    return jnp.where(v >= 0, v, _NEG * v)


def _bn_act(v, g, b):
    m = jnp.mean(v, axis=0, keepdims=True)
    var = jnp.mean((v - m) ** 2, axis=0, keepdims=True)
    return _leaky((v - m) / jnp.sqrt(var + _EPS) * g + b)


def _unary1_body(x_ref, w_ref, g_ref, b_ref, o_ref):
    y = jnp.dot(x_ref[...], w_ref[...], preferred_element_type=jnp.float32)
    o_ref[...] = _bn_act(y, g_ref[...], b_ref[...])


def _knn_agg_body(q_rows, npad, big, xyzq_ref, xyzt_ref, rhi_ref,
                  kptst_ref, wkp_ref, o_ref):
    qx = xyzq_ref[:, 0:1]
    qy = xyzq_ref[:, 1:2]
    qz = xyzq_ref[:, 2:3]
    sx = xyzt_ref[0:1, :]
    sy = xyzt_ref[1:2, :]
    sz = xyzt_ref[2:3, :]
    dx = qx - sx
    dy = qy - sy
    dz = qz - sz
    d2 = dx * dx + dy * dy + dz * dz
    iota = jax.lax.broadcasted_iota(jnp.int32, (q_rows, npad), 1)
    kx = kptst_ref[0:1, :]
    ky = kptst_ref[1:2, :]
    kz = kptst_ref[2:3, :]
    accs = tuple(jnp.zeros((q_rows, 32), jnp.float32) for _ in range(_KPTS))

    def step(_, carry):
        d2, accs = carry
        m = jnp.min(d2, axis=1, keepdims=True)
        amin = jnp.min(jnp.where(d2 == m, iota, npad), axis=1, keepdims=True)
        eq = iota == amin
        selx = jnp.sum(jnp.where(eq, sx, 0.0), axis=1, keepdims=True)
        sely = jnp.sum(jnp.where(eq, sy, 0.0), axis=1, keepdims=True)
        selz = jnp.sum(jnp.where(eq, sz, 0.0), axis=1, keepdims=True)
        feat = jnp.dot(eq.astype(jnp.bfloat16), rhi_ref[...],
                       preferred_element_type=jnp.float32)
        d2 = jnp.where(eq, big, d2)
        rx = selx - qx
        ry = sely - qy
        rz = selz - qz
        sq = (rx - kx) ** 2 + (ry - ky) ** 2 + (rz - kz) ** 2
        w = jnp.maximum(1.0 - jnp.sqrt(sq + 1e-12) / _INFL, 0.0)
        accs = tuple(accs[k] + w[:, k:k + 1] * feat for k in range(_KPTS))
        return d2, accs

    _, accs = jax.lax.fori_loop(0, _KNN, step, (d2, accs))
    out = jnp.zeros((q_rows, 32), jnp.float32)
    for k in range(_KPTS):
        out = out + jnp.dot(accs[k], wkp_ref[k],
                            preferred_element_type=jnp.float32)
    o_ref[...] = out


def _post_body(a_ref, gk_ref, bk_ref, w2_ref, g2_ref, b2_ref, xs_ref, o_ref):
    h = _bn_act(a_ref[...], gk_ref[...], bk_ref[...])
    y = jnp.dot(h, w2_ref[...], preferred_element_type=jnp.float32)
    o_ref[...] = _bn_act(y, g2_ref[...], b2_ref[...]) + xs_ref[...]


@jax.jit
def kernel(pos, x, W1, g1, b1, kpts, Wkp, gk, bk, W2, g2, b2):
    n, d_in = x.shape
    d2ch = W1.shape[1]
    xyz = pos[:, 1:4]

    h1 = pl.pallas_call(
        _unary1_body,
        out_shape=jax.ShapeDtypeStruct((n, d2ch), jnp.float32),
    )(x, W1, g1.reshape(1, -1), b1.reshape(1, -1))

    q_rows = 256
    npad = ((n + q_rows - 1) // q_rows) * q_rows
    pad = npad - n
    padv = 1e6
    xyz_pad = jnp.concatenate(
        [xyz, jnp.full((pad, 3), padv, jnp.float32)], axis=0)
    h1_pad = jnp.concatenate([h1, jnp.zeros((pad, d2ch), jnp.float32)], axis=0)
    r_hi = h1_pad.astype(jnp.bfloat16)
    xyzt = xyz_pad.T
    kptst = kpts.T
    grid = npad // q_rows

    body = functools.partial(_knn_agg_body, q_rows, npad, 1e30)
    agg = pl.pallas_call(
        body,
        grid=(grid,),
        in_specs=[
            pl.BlockSpec((q_rows, 3), lambda i: (i, 0)),
            pl.BlockSpec((3, npad), lambda i: (0, 0)),
            pl.BlockSpec((npad, d2ch), lambda i: (0, 0)),
            pl.BlockSpec((3, _KPTS), lambda i: (0, 0)),
            pl.BlockSpec((_KPTS, d2ch, d2ch), lambda i: (0, 0, 0)),
        ],
        out_specs=pl.BlockSpec((q_rows, d2ch), lambda i: (i, 0)),
        out_shape=jax.ShapeDtypeStruct((npad, d2ch), jnp.float32),
        compiler_params=pltpu.CompilerParams(
            dimension_semantics=("arbitrary",)),
    )(xyz_pad, xyzt, r_hi, kptst, Wkp)
    agg = agg[:n]

    out = pl.pallas_call(
        _post_body,
        out_shape=jax.ShapeDtypeStruct((n, d_in), jnp.float32),
    )(agg, gk.reshape(1, -1), bk.reshape(1, -1), W2, g2.reshape(1, -1),
      b2.reshape(1, -1), x)
    return out


# speedup vs baseline: 3.7732x; 1.4803x over previous
import functools

import jax
import jax.numpy as jnp
from jax.experimental import pallas as pl
from jax.experimental.pallas import tpu as pltpu

_EPS = 1e-5
_NEG = 0.1
_INFL = 0.08
_KNN = 16
_KPTS = 15


def _leaky(v):
    return jnp.where(v >= 0, v, _NEG * v)


def _bn_act(v, g, b):
    m = jnp.mean(v, axis=0, keepdims=True)
    var = jnp.mean((v - m) ** 2, axis=0, keepdims=True)
    return _leaky((v - m) / jnp.sqrt(var + _EPS) * g + b)


def _unary1_body(x_ref, w_ref, g_ref, b_ref, xyz_ref, kptst_ref,
                 h_ref, p_ref, phi_ref, plo_ref):
    y = jnp.dot(x_ref[...], w_ref[...], preferred_element_type=jnp.float32)
    h_ref[...] = _bn_act(y, g_ref[...], b_ref[...])
    p = (xyz_ref[:, 0:1] * kptst_ref[0:1, :]
         + xyz_ref[:, 1:2] * kptst_ref[1:2, :]
         + xyz_ref[:, 2:3] * kptst_ref[2:3, :])
    p_ref[...] = p
    phi = p.astype(jnp.bfloat16)
    phi_ref[...] = phi
    plo_ref[...] = (p - phi.astype(jnp.float32)).astype(jnp.bfloat16)


def _knn_agg_body(q_rows, npad, big, xyzq_ref, xyzt_ref, t_ref, pq_ref,
                  ck_ref, wkp_ref, o_ref):
    qx = xyzq_ref[:, 0:1]
    qy = xyzq_ref[:, 1:2]
    qz = xyzq_ref[:, 2:3]
    dx = qx - xyzt_ref[0:1, :]
    dy = qy - xyzt_ref[1:2, :]
    dz = qz - xyzt_ref[2:3, :]
    d2 = dx * dx + dy * dy + dz * dz
    iota = jax.lax.broadcasted_iota(jnp.int32, (q_rows, npad), 1)
    two_pq_ck = 2.0 * pq_ref[...] + ck_ref[...]
    accs = tuple(jnp.zeros((q_rows, 32), jnp.float32) for _ in range(_KPTS))

    def step(_, carry):
        d2, accs = carry
        m = jnp.min(d2, axis=1, keepdims=True)
        amin = jnp.min(jnp.where(d2 == m, iota, npad), axis=1, keepdims=True)
        eqb = (iota == amin).astype(jnp.bfloat16)
        g = jnp.dot(eqb, t_ref[...], preferred_element_type=jnp.float32)
        d2 = jnp.where(iota == amin, big, d2)
        pg = g[:, 0:_KPTS] + g[:, _KPTS:2 * _KPTS]
        feat = g[:, 2 * _KPTS:2 * _KPTS + 32]
        sq = jnp.maximum(m - 2.0 * pg + two_pq_ck, 0.0)
        w = jnp.maximum(1.0 - jnp.sqrt(sq + 1e-12) / _INFL, 0.0)
        accs = tuple(accs[k] + w[:, k:k + 1] * feat for k in range(_KPTS))
        return d2, accs

    _, accs = jax.lax.fori_loop(0, _KNN, step, (d2, accs))
    out = jnp.zeros((q_rows, 32), jnp.float32)
    for k in range(_KPTS):
        out = out + jnp.dot(accs[k], wkp_ref[k],
                            preferred_element_type=jnp.float32)
    o_ref[...] = out


def _post_body(a_ref, gk_ref, bk_ref, w2_ref, g2_ref, b2_ref, xs_ref, o_ref):
    h = _bn_act(a_ref[...], gk_ref[...], bk_ref[...])
    y = jnp.dot(h, w2_ref[...], preferred_element_type=jnp.float32)
    o_ref[...] = _bn_act(y, g2_ref[...], b2_ref[...]) + xs_ref[...]


@jax.jit
def kernel(pos, x, W1, g1, b1, kpts, Wkp, gk, bk, W2, g2, b2):
    n, d_in = x.shape
    d2ch = W1.shape[1]
    xyz = pos[:, 1:4]
    kptst = kpts.T

    h1, p_proj, p_hi, p_lo = pl.pallas_call(
        _unary1_body,
        out_shape=[jax.ShapeDtypeStruct((n, d2ch), jnp.float32),
                   jax.ShapeDtypeStruct((n, _KPTS), jnp.float32),
                   jax.ShapeDtypeStruct((n, _KPTS), jnp.bfloat16),
                   jax.ShapeDtypeStruct((n, _KPTS), jnp.bfloat16)],
    )(x, W1, g1.reshape(1, -1), b1.reshape(1, -1), xyz, kptst)

    q_rows = 256
    npad = ((n + q_rows - 1) // q_rows) * q_rows
    pad = npad - n
    padv = 1e6
    xyz_pad = jnp.concatenate(
        [xyz, jnp.full((pad, 3), padv, jnp.float32)], axis=0)
    p_pad = jnp.concatenate([p_proj, jnp.zeros((pad, _KPTS), jnp.float32)],
                            axis=0)
    zb = jnp.zeros((pad, _KPTS), jnp.bfloat16)
    t_mat = jnp.concatenate([
        jnp.concatenate([p_hi, zb], axis=0),
        jnp.concatenate([p_lo, zb], axis=0),
        jnp.concatenate([h1.astype(jnp.bfloat16),
                         jnp.zeros((pad, d2ch), jnp.bfloat16)], axis=0),
    ], axis=1)
    xyzt = xyz_pad.T
    ck = jnp.sum(kpts * kpts, axis=1).reshape(1, -1)
    grid = npad // q_rows

    body = functools.partial(_knn_agg_body, q_rows, npad, 1e30)
    agg = pl.pallas_call(
        body,
        grid=(grid,),
        in_specs=[
            pl.BlockSpec((q_rows, 3), lambda i: (i, 0)),
            pl.BlockSpec((3, npad), lambda i: (0, 0)),
            pl.BlockSpec((npad, 2 * _KPTS + 32), lambda i: (0, 0)),
            pl.BlockSpec((q_rows, _KPTS), lambda i: (i, 0)),
            pl.BlockSpec((1, _KPTS), lambda i: (0, 0)),
            pl.BlockSpec((_KPTS, d2ch, d2ch), lambda i: (0, 0, 0)),
        ],
        out_specs=pl.BlockSpec((q_rows, d2ch), lambda i: (i, 0)),
        out_shape=jax.ShapeDtypeStruct((npad, d2ch), jnp.float32),
        compiler_params=pltpu.CompilerParams(
            dimension_semantics=("arbitrary",)),
    )(xyz_pad, xyzt, t_mat, p_pad, ck, Wkp)
    agg = agg[:n]

    out = pl.pallas_call(
        _post_body,
        out_shape=jax.ShapeDtypeStruct((n, d_in), jnp.float32),
    )(agg, gk.reshape(1, -1), bk.reshape(1, -1), W2, g2.reshape(1, -1),
      b2.reshape(1, -1), x)
    return out


# speedup vs baseline: 3.7912x; 1.0048x over previous
import functools

import jax
import jax.numpy as jnp
from jax.experimental import pallas as pl
from jax.experimental.pallas import tpu as pltpu

_EPS = 1e-5
_NEG = 0.1
_INFL = 0.08
_KNN = 16
_KPTS = 15


def _leaky(v):
    return jnp.where(v >= 0, v, _NEG * v)


def _bn_act(v, g, b):
    m = jnp.mean(v, axis=0, keepdims=True)
    var = jnp.mean((v - m) ** 2, axis=0, keepdims=True)
    return _leaky((v - m) / jnp.sqrt(var + _EPS) * g + b)


def _unary1_body(x_ref, w_ref, g_ref, b_ref, xyz_ref, kptst_ref,
                 h_ref, p_ref, phi_ref, plo_ref):
    y = jnp.dot(x_ref[...], w_ref[...], preferred_element_type=jnp.float32)
    h_ref[...] = _bn_act(y, g_ref[...], b_ref[...])
    p = (xyz_ref[:, 0:1] * kptst_ref[0:1, :]
         + xyz_ref[:, 1:2] * kptst_ref[1:2, :]
         + xyz_ref[:, 2:3] * kptst_ref[2:3, :])
    p_ref[...] = p
    phi = p.astype(jnp.bfloat16)
    phi_ref[...] = phi
    plo_ref[...] = (p - phi.astype(jnp.float32)).astype(jnp.bfloat16)


def _knn_agg_body(q_rows, npad, big, xyzq_ref, xyzt_ref, t_ref, pq_ref,
                  ck_ref, wkp_ref, o_ref):
    qx = xyzq_ref[:, 0:1]
    qy = xyzq_ref[:, 1:2]
    qz = xyzq_ref[:, 2:3]
    dx = qx - xyzt_ref[0:1, :]
    dy = qy - xyzt_ref[1:2, :]
    dz = qz - xyzt_ref[2:3, :]
    d2 = dx * dx + dy * dy + dz * dz
    iota = jax.lax.broadcasted_iota(jnp.int32, (q_rows, npad), 1)
    two_pq_ck = 2.0 * pq_ref[...] + ck_ref[...]
    accs = tuple(jnp.zeros((q_rows, 32), jnp.float32) for _ in range(_KPTS))

    def step(_, carry):
        d2, accs = carry
        m = jnp.min(d2, axis=1, keepdims=True)
        amin = jnp.min(jnp.where(d2 == m, iota, npad), axis=1, keepdims=True)
        eqf = (iota == amin).astype(jnp.float32)
        g = jnp.dot(eqf, t_ref[...], preferred_element_type=jnp.float32)
        d2 = jnp.where(iota == amin, big, d2)
        pg = g[:, 0:_KPTS] + g[:, _KPTS:2 * _KPTS]
        feat = g[:, 2 * _KPTS:2 * _KPTS + 32]
        sq = jnp.maximum(m - 2.0 * pg + two_pq_ck, 0.0)
        w = jnp.maximum(1.0 - jnp.sqrt(sq + 1e-12) / _INFL, 0.0)
        accs = tuple(accs[k] + w[:, k:k + 1] * feat for k in range(_KPTS))
        return d2, accs

    _, accs = jax.lax.fori_loop(0, _KNN, step, (d2, accs))
    out = jnp.zeros((q_rows, 32), jnp.float32)
    for k in range(_KPTS):
        out = out + jnp.dot(accs[k], wkp_ref[k],
                            preferred_element_type=jnp.float32)
    o_ref[...] = out


def _post_body(a_ref, gk_ref, bk_ref, w2_ref, g2_ref, b2_ref, xs_ref, o_ref):
    h = _bn_act(a_ref[...], gk_ref[...], bk_ref[...])
    y = jnp.dot(h, w2_ref[...], preferred_element_type=jnp.float32)
    o_ref[...] = _bn_act(y, g2_ref[...], b2_ref[...]) + xs_ref[...]


@jax.jit
def kernel(pos, x, W1, g1, b1, kpts, Wkp, gk, bk, W2, g2, b2):
    n, d_in = x.shape
    d2ch = W1.shape[1]
    xyz = pos[:, 1:4]
    kptst = kpts.T

    h1, p_proj, p_hi, p_lo = pl.pallas_call(
        _unary1_body,
        out_shape=[jax.ShapeDtypeStruct((n, d2ch), jnp.float32),
                   jax.ShapeDtypeStruct((n, _KPTS), jnp.float32),
                   jax.ShapeDtypeStruct((n, _KPTS), jnp.bfloat16),
                   jax.ShapeDtypeStruct((n, _KPTS), jnp.bfloat16)],
    )(x, W1, g1.reshape(1, -1), b1.reshape(1, -1), xyz, kptst)

    q_rows = 256
    npad = ((n + q_rows - 1) // q_rows) * q_rows
    pad = npad - n
    padv = 1e6
    xyz_pad = jnp.concatenate(
        [xyz, jnp.full((pad, 3), padv, jnp.float32)], axis=0)
    p_pad = jnp.concatenate([p_proj, jnp.zeros((pad, _KPTS), jnp.float32)],
                            axis=0)
    zf = jnp.zeros((pad, _KPTS), jnp.float32)
    h1_pad = jnp.concatenate([h1, jnp.zeros((pad, d2ch), jnp.float32)], axis=0)
    t_mat = jnp.concatenate([
        jnp.concatenate([p_hi.astype(jnp.float32), zf], axis=0),
        jnp.concatenate([p_lo.astype(jnp.float32), zf], axis=0),
        h1_pad,
    ], axis=1)
    xyzt = xyz_pad.T
    ck = jnp.sum(kpts * kpts, axis=1).reshape(1, -1)
    grid = npad // q_rows

    body = functools.partial(_knn_agg_body, q_rows, npad, 1e30)
    agg = pl.pallas_call(
        body,
        grid=(grid,),
        in_specs=[
            pl.BlockSpec((q_rows, 3), lambda i: (i, 0)),
            pl.BlockSpec((3, npad), lambda i: (0, 0)),
            pl.BlockSpec((npad, 2 * _KPTS + 32), lambda i: (0, 0)),
            pl.BlockSpec((q_rows, _KPTS), lambda i: (i, 0)),
            pl.BlockSpec((1, _KPTS), lambda i: (0, 0)),
            pl.BlockSpec((_KPTS, d2ch, d2ch), lambda i: (0, 0, 0)),
        ],
        out_specs=pl.BlockSpec((q_rows, d2ch), lambda i: (i, 0)),
        out_shape=jax.ShapeDtypeStruct((npad, d2ch), jnp.float32),
        compiler_params=pltpu.CompilerParams(
            dimension_semantics=("arbitrary",)),
    )(xyz_pad, xyzt, t_mat, p_pad, ck, Wkp)
    agg = agg[:n]

    out = pl.pallas_call(
        _post_body,
        out_shape=jax.ShapeDtypeStruct((n, d_in), jnp.float32),
    )(agg, gk.reshape(1, -1), bk.reshape(1, -1), W2, g2.reshape(1, -1),
      b2.reshape(1, -1), x)
    return out


# speedup vs baseline: 3.8095x; 1.0048x over previous
import functools

import jax
import jax.numpy as jnp
from jax import lax
from jax.experimental import pallas as pl
from jax.experimental.pallas import tpu as pltpu
from jax.experimental.pallas import tpu_sc as plsc

_EPS = 1e-5
_NEG = 0.1
_INFL = 0.08
_KNN = 16
_KPTS = 15
_TROW = 128


def _leaky(v):
    return jnp.where(v >= 0, v, _NEG * v)


def _bn_act(v, g, b):
    m = jnp.mean(v, axis=0, keepdims=True)
    var = jnp.mean((v - m) ** 2, axis=0, keepdims=True)
    return _leaky((v - m) / jnp.sqrt(var + _EPS) * g + b)


def _unary1_body(x_ref, w_ref, g_ref, b_ref, xyz_ref, kptst_ref,
                 h_ref, p_ref):
    y = jnp.dot(x_ref[...], w_ref[...], preferred_element_type=jnp.float32)
    h_ref[...] = _bn_act(y, g_ref[...], b_ref[...])
    p_ref[...] = (xyz_ref[:, 0:1] * kptst_ref[0:1, :]
                  + xyz_ref[:, 1:2] * kptst_ref[1:2, :]
                  + xyz_ref[:, 2:3] * kptst_ref[2:3, :])


def _knn_body(q_rows, npad, big, xyzq_ref, xyzt_ref, oi_ref, om_ref):
    qx = xyzq_ref[:, 0:1]
    qy = xyzq_ref[:, 1:2]
    qz = xyzq_ref[:, 2:3]
    dx = qx - xyzt_ref[0:1, :]
    dy = qy - xyzt_ref[1:2, :]
    dz = qz - xyzt_ref[2:3, :]
    d2 = dx * dx + dy * dy + dz * dz
    iota = jax.lax.broadcasted_iota(jnp.int32, (q_rows, npad), 1)
    lane16 = jax.lax.broadcasted_iota(jnp.int32, (q_rows, _KNN), 1)
    idxacc = jnp.zeros((q_rows, _KNN), jnp.int32)
    macc = jnp.zeros((q_rows, _KNN), jnp.float32)

    def step(t, carry):
        d2, idxacc, macc = carry
        m = jnp.min(d2, axis=1, keepdims=True)
        amin = jnp.min(jnp.where(d2 == m, iota, npad), axis=1, keepdims=True)
        d2 = jnp.where(iota == amin, big, d2)
        idxacc = jnp.where(lane16 == t, amin, idxacc)
        macc = jnp.where(lane16 == t, m, macc)
        return d2, idxacc, macc

    _, idxacc, macc = jax.lax.fori_loop(0, _KNN, step, (d2, idxacc, macc))
    oi_ref[...] = idxacc
    om_ref[...] = macc


def _agg_body(q_rows, xyzq_dummy, fr_ref, msel_ref, pq_ref, ck_ref, wkp_ref,
              o_ref):
    two_pq_ck = 2.0 * pq_ref[...] + ck_ref[...]
    accs = [jnp.zeros((q_rows, 32), jnp.float32) for _ in range(_KPTS)]
    for t in range(_KNN):
        feat = fr_ref[:, t * _TROW:t * _TROW + 32]
        pg = fr_ref[:, t * _TROW + 32:t * _TROW + 32 + _KPTS]
        m = msel_ref[:, t:t + 1]
        sq = jnp.maximum(m - 2.0 * pg + two_pq_ck, 0.0)
        w = jnp.maximum(1.0 - jnp.sqrt(sq + 1e-12) / _INFL, 0.0)
        for k in range(_KPTS):
            accs[k] = accs[k] + w[:, k:k + 1] * feat
    out = jnp.zeros((q_rows, 32), jnp.float32)
    for k in range(_KPTS):
        out = out + jnp.dot(accs[k], wkp_ref[k],
                            preferred_element_type=jnp.float32)
    o_ref[...] = out


def _post_body(a_ref, gk_ref, bk_ref, w2_ref, g2_ref, b2_ref, xs_ref, o_ref):
    h = _bn_act(a_ref[...], gk_ref[...], bk_ref[...])
    y = jnp.dot(h, w2_ref[...], preferred_element_type=jnp.float32)
    o_ref[...] = _bn_act(y, g2_ref[...], b2_ref[...]) + xs_ref[...]


@jax.jit
def kernel(pos, x, W1, g1, b1, kpts, Wkp, gk, bk, W2, g2, b2):
    n, d_in = x.shape
    d2ch = W1.shape[1]
    xyz = pos[:, 1:4]
    kptst = kpts.T

    h1, p_proj = pl.pallas_call(
        _unary1_body,
        out_shape=[jax.ShapeDtypeStruct((n, d2ch), jnp.float32),
                   jax.ShapeDtypeStruct((n, _KPTS), jnp.float32)],
    )(x, W1, g1.reshape(1, -1), b1.reshape(1, -1), xyz, kptst)

    q_rows = 256
    npad = ((n + q_rows - 1) // q_rows) * q_rows
    pad = npad - n
    padv = 1e6
    xyz_pad = jnp.concatenate(
        [xyz, jnp.full((pad, 3), padv, jnp.float32)], axis=0)
    p_pad = jnp.concatenate([p_proj, jnp.zeros((pad, _KPTS), jnp.float32)],
                            axis=0)
    xyzt = xyz_pad.T
    ck = jnp.sum(kpts * kpts, axis=1).reshape(1, -1)
    grid = npad // q_rows

    knn = functools.partial(_knn_body, q_rows, npad, 1e30)
    idx, msel = pl.pallas_call(
        knn,
        grid=(grid,),
        in_specs=[
            pl.BlockSpec((q_rows, 3), lambda i: (i, 0)),
            pl.BlockSpec((3, npad), lambda i: (0, 0)),
        ],
        out_specs=[pl.BlockSpec((q_rows, _KNN), lambda i: (i, 0)),
                   pl.BlockSpec((q_rows, _KNN), lambda i: (i, 0))],
        out_shape=[jax.ShapeDtypeStruct((npad, _KNN), jnp.int32),
                   jax.ShapeDtypeStruct((npad, _KNN), jnp.float32)],
        compiler_params=pltpu.CompilerParams(
            dimension_semantics=("arbitrary",)),
    )(xyz_pad, xyzt)

    t2 = jnp.concatenate([
        jnp.concatenate([h1, jnp.zeros((pad, d2ch), jnp.float32)], axis=0),
        p_pad,
        jnp.zeros((npad, _TROW - d2ch - _KPTS), jnp.float32),
    ], axis=1)
    idx_flat = idx.reshape(-1)
    n_edges = npad * _KNN

    info = plsc.get_sparse_core_info()
    nc, ns = info.num_cores, info.num_subcores
    nw = nc * ns
    per_w = n_edges // nw
    ch = 128
    n_ch = per_w // ch
    mesh = plsc.VectorSubcoreMesh(core_axis_name="c", subcore_axis_name="s")

    @functools.partial(
        pl.kernel, mesh=mesh,
        out_type=jax.ShapeDtypeStruct((n_edges, _TROW), jnp.float32),
        scratch_types=[
            pltpu.VMEM((ch,), jnp.int32),
            pltpu.VMEM((ch, _TROW), jnp.float32),
            pltpu.SemaphoreType.DMA,
        ],
    )
    def sc_gather(table_hbm, idx_hbm, out_hbm, idx_v, rows_v, sem):
        wid = lax.axis_index("s") * nc + lax.axis_index("c")
        base0 = wid * per_w

        def body(c, carry):
            base = base0 + c * ch
            pltpu.sync_copy(idx_hbm.at[pl.ds(base, ch)], idx_v)
            pltpu.async_copy(table_hbm.at[idx_v], rows_v, sem).wait()
            pltpu.sync_copy(rows_v, out_hbm.at[pl.ds(base, ch)])
            return carry

        lax.fori_loop(0, n_ch, body, 0)

    f_rows = sc_gather(t2, idx_flat)
    fr = f_rows.reshape(npad, _KNN * _TROW)

    agg_fn = functools.partial(_agg_body, q_rows)
    agg = pl.pallas_call(
        agg_fn,
        grid=(grid,),
        in_specs=[
            pl.BlockSpec((q_rows, 3), lambda i: (i, 0)),
            pl.BlockSpec((q_rows, _KNN * _TROW), lambda i: (i, 0)),
            pl.BlockSpec((q_rows, _KNN), lambda i: (i, 0)),
            pl.BlockSpec((q_rows, _KPTS), lambda i: (i, 0)),
            pl.BlockSpec((1, _KPTS), lambda i: (0, 0)),
            pl.BlockSpec((_KPTS, d2ch, d2ch), lambda i: (0, 0, 0)),
        ],
        out_specs=pl.BlockSpec((q_rows, d2ch), lambda i: (i, 0)),
        out_shape=jax.ShapeDtypeStruct((npad, d2ch), jnp.float32),
        compiler_params=pltpu.CompilerParams(
            dimension_semantics=("arbitrary",)),
    )(xyz_pad, fr, msel, p_pad, ck, Wkp)
    agg = agg[:n]

    out = pl.pallas_call(
        _post_body,
        out_shape=jax.ShapeDtypeStruct((n, d_in), jnp.float32),
    )(agg, gk.reshape(1, -1), bk.reshape(1, -1), W2, g2.reshape(1, -1),
      b2.reshape(1, -1), x)
    return out
